# baseline (speedup 1.0000x reference)
import jax
import jax.numpy as jnp
from jax import lax
from jax.experimental import pallas as pl
from jax.experimental.pallas import tpu as pltpu
from jax.experimental.pallas import tpu_sc as plsc

N_ROWS = 8192
N_COLS = 4096
BLK = 8
KEEP = 4

TC_ROWS = 8192 - 256
SC_ROWS = N_ROWS - TC_ROWS
TC_TILE = 256

NC = 2
NS = 16
NW = NC * NS
PER_W = SC_ROWS * N_COLS // NW
CHUNK = 16384
CW = CHUNK // 8
N_CHUNKS = PER_W // CHUNK
PANEL = 128
N_PANELS = CHUNK // PANEL


def _sc_body(x2_hbm, s2_hbm, o2_hbm,
             xb0, xb1, sb0, sb1, ob0, ob1,
             sx0, sx1, ss0, ss1, so0, so1):
    x_hbm = x2_hbm.reshape(N_ROWS // 8, 8, N_COLS)
    s_hbm = s2_hbm.reshape(N_ROWS // 8, 8, N_COLS)
    o_hbm = o2_hbm.reshape(SC_ROWS // 8, 8, N_COLS)
    wid = lax.axis_index("s") * NC + lax.axis_index("c")
    base_rc = wid * (N_CHUNKS // 2)

    lane8 = lax.iota(jnp.int32, 16) * 8

    xbufs, sbufs, obufs = (xb0, xb1), (sb0, sb1), (ob0, ob1)
    sxs, sss, sos = (sx0, sx1), (ss0, ss1), (so0, so1)

    def in_copies(c, b):
        rc = TC_ROWS // 8 + base_rc + (c >> 1)
        ch = (c & 1) * CW
        return (
            pltpu.make_async_copy(
                x_hbm.at[rc, :, pl.ds(ch, CW)], xbufs[b], sxs[b]),
            pltpu.make_async_copy(
                s_hbm.at[rc, :, pl.ds(ch, CW)], sbufs[b], sss[b]),
        )

    def out_copy(c, b):
        rc = base_rc + (c >> 1)
        ch = (c & 1) * CW
        return pltpu.make_async_copy(
            obufs[b], o_hbm.at[rc, :, pl.ds(ch, CW)], sos[b])

    def do_chunk(c, b):
        xb, sb, ob = xbufs[b], sbufs[b], obufs[b]

        @pl.when(c + 1 < N_CHUNKS)
        def _():
            nx, ns = in_copies(c + 1, 1 - b)
            nx.start()
            ns.start()

        cx, cs = in_copies(c, b)
        cx.wait()
        cs.wait()

        @pl.when(c >= 2)
        def _():
            out_copy(c - 2, b).wait()

        idx = [lane8 + k for k in range(BLK)]

        @plsc.parallel_loop(0, N_PANELS, unroll=2)
        def do_panel(p):
            r = p >> 4
            pbase = (p & 15) * PANEL
            sp = sb.at[r, pl.ds(pbase, PANEL)]
            xp = xb.at[r, pl.ds(pbase, PANEL)]
            op = ob.at[r, pl.ds(pbase, PANEL)]
            s = [plsc.load_gather(sp, [idx[k]]) for k in range(BLK)]
            terms = []
            for i in range(BLK):
                for j in range(i + 1, BLK):
                    kc = (1 << (4 * i)) - (1 << (4 * j))
                    terms.append(
                        jnp.where(s[j] < s[i], jnp.int32(kc), jnp.int32(0))
                    )
            while len(terms) > 1:
                terms = [
                    terms[t] + terms[t + 1] if t + 1 < len(terms) else terms[t]
                    for t in range(0, len(terms), 2)
                ]
            init = sum(k << (4 * k) for k in range(BLK))
            packed = jnp.full((16,), init, jnp.int32) + terms[0]
            zeros = jnp.zeros((16,), jnp.float32)
            for k in range(BLK):
                op[pl.ds(k * 16, 16)] = zeros
            for k in range(BLK):
                xk = plsc.load_gather(xp, [idx[k]])
                keep = (packed & (KEEP << (4 * k))) != 0
                plsc.store_scatter(op, [idx[k]], xk, mask=keep)

        out_copy(c, b).start()

    nx, ns = in_copies(0, 0)
    nx.start()
    ns.start()

    def pair_body(i, _):
        do_chunk(2 * i, 0)
        do_chunk(2 * i + 1, 1)
        return ()

    lax.fori_loop(0, N_CHUNKS // 2, pair_body, ())
    out_copy(N_CHUNKS - 2, 0).wait()
    out_copy(N_CHUNKS - 1, 1).wait()


def _tc_body(x_ref, s_ref, o_ref):
    s = s_ref[...]
    n = s.shape[1]
    pos = lax.broadcasted_iota(jnp.int32, s.shape, 1) & 7
    cnt = jnp.zeros(s.shape, jnp.int32)
    for r in range(1, BLK):
        fwd = pltpu.roll(s, n - r, 1)
        bwd = pltpu.roll(s, BLK - r, 1)
        wrap = (pos + r) >= BLK
        partner = jnp.where(wrap, bwd, fwd)
        beats = (partner < s) | ((partner == s) & wrap)
        cnt = cnt + beats.astype(jnp.int32)
    o_ref[...] = jnp.where(cnt >= KEEP, x_ref[...], 0.0)


@jax.jit
def _sparsify(x, s):
    mesh = plsc.VectorSubcoreMesh(core_axis_name="c", subcore_axis_name="s")
    sc_run = pl.kernel(
        _sc_body,
        mesh=mesh,
        out_type=jax.ShapeDtypeStruct((SC_ROWS, N_COLS), jnp.float32),
        scratch_types=(
            [pltpu.VMEM((8, CW), jnp.float32)] * 6
            + [pltpu.SemaphoreType.DMA] * 6
        ),
        compiler_params=pltpu.CompilerParams(needs_layout_passes=False),
    )
    sc_out = sc_run(x, s)
    tc_out = pl.pallas_call(
        _tc_body,
        grid=(TC_ROWS // TC_TILE,),
        in_specs=[pl.BlockSpec((TC_TILE, N_COLS), lambda i: (i, 0))] * 2,
        out_specs=pl.BlockSpec((TC_TILE, N_COLS), lambda i: (i, 0)),
        out_shape=jax.ShapeDtypeStruct((TC_ROWS, N_COLS), jnp.float32),
    )(x, s)
    return jnp.concatenate([tc_out, sc_out], axis=0)


def kernel(x, score):
    return _sparsify(x, score)


# speedup vs baseline: 3.6984x; 3.6984x over previous
import jax
import jax.numpy as jnp
from jax import lax
from jax.experimental import pallas as pl
from jax.experimental.pallas import tpu as pltpu
from jax.experimental.pallas import tpu_sc as plsc

N_ROWS = 8192
N_COLS = 4096
BLK = 8
KEEP = 4

NC = 2
NS = 16
NW = NC * NS
PER_W = N_ROWS * N_COLS // NW
CHUNK = 16384
CW = CHUNK // 8
N_CHUNKS = PER_W // CHUNK
PANEL = 128
N_PANELS = CHUNK // PANEL


def _sc_body(x2_hbm, s2_hbm, o2_hbm,
             sb0, sb1, ob0, ob1,
             sx0, sx1, ss0, ss1, so0, so1):
    x_hbm = x2_hbm.reshape(N_ROWS // 8, 8, N_COLS)
    s_hbm = s2_hbm.reshape(N_ROWS // 8, 8, N_COLS)
    o_hbm = o2_hbm.reshape(N_ROWS // 8, 8, N_COLS)
    wid = lax.axis_index("s") * NC + lax.axis_index("c")
    base_rc = wid * (N_CHUNKS // 2)

    lane8 = lax.iota(jnp.int32, 16) * 8

    sbufs, obufs = (sb0, sb1), (ob0, ob1)
    sxs, sss, sos = (sx0, sx1), (ss0, ss1), (so0, so1)

    def in_copies(c, b):
        rc = base_rc + (c >> 1)
        ch = (c & 1) * CW
        return (
            pltpu.make_async_copy(
                x_hbm.at[rc, :, pl.ds(ch, CW)], obufs[b], sxs[b]),
            pltpu.make_async_copy(
                s_hbm.at[rc, :, pl.ds(ch, CW)], sbufs[b], sss[b]),
        )

    def out_copy(c, b):
        rc = base_rc + (c >> 1)
        ch = (c & 1) * CW
        return pltpu.make_async_copy(
            obufs[b], o_hbm.at[rc, :, pl.ds(ch, CW)], sos[b])

    def do_chunk(c, b):
        sb, ob = sbufs[b], obufs[b]

        @pl.when(c + 1 < N_CHUNKS)
        def _():
            @pl.when(c >= 1)
            def _():
                out_copy(c - 1, 1 - b).wait()
            nx, ns = in_copies(c + 1, 1 - b)
            nx.start()
            ns.start()

        cx, cs = in_copies(c, b)
        cx.wait()
        cs.wait()

        idx = [lane8 + k for k in range(BLK)]

        @plsc.parallel_loop(0, N_PANELS, unroll=2)
        def do_panel(p):
            r = p >> 4
            pbase = (p & 15) * PANEL
            sp = sb.at[r, pl.ds(pbase, PANEL)]
            op = ob.at[r, pl.ds(pbase, PANEL)]
            s = [plsc.load_gather(sp, [idx[k]]) for k in range(BLK)]
            terms = []
            for i in range(BLK):
                for j in range(i + 1, BLK):
                    kc = (1 << (4 * i)) - (1 << (4 * j))
                    terms.append(
                        jnp.where(s[j] < s[i], jnp.int32(kc), jnp.int32(0))
                    )
            while len(terms) > 1:
                terms = [
                    terms[t] + terms[t + 1] if t + 1 < len(terms) else terms[t]
                    for t in range(0, len(terms), 2)
                ]
            init = sum(k << (4 * k) for k in range(BLK))
            packed = jnp.full((16,), init, jnp.int32) + terms[0]
            zeros = jnp.zeros((16,), jnp.float32)
            for k in range(BLK):
                drop = (packed & (KEEP << (4 * k))) == 0
                plsc.store_scatter(op, [idx[k]], zeros, mask=drop)

        out_copy(c, b).start()

    nx, ns = in_copies(0, 0)
    nx.start()
    ns.start()

    def pair_body(i, _):
        do_chunk(2 * i, 0)
        do_chunk(2 * i + 1, 1)
        return ()

    lax.fori_loop(0, N_CHUNKS // 2, pair_body, ())
    out_copy(N_CHUNKS - 2, 0).wait()
    out_copy(N_CHUNKS - 1, 1).wait()


@jax.jit
def _sparsify(x, s):
    mesh = plsc.VectorSubcoreMesh(core_axis_name="c", subcore_axis_name="s")
    run = pl.kernel(
        _sc_body,
        mesh=mesh,
        out_type=jax.ShapeDtypeStruct((N_ROWS, N_COLS), jnp.float32),
        scratch_types=(
            [pltpu.VMEM((8, CW), jnp.float32)] * 4
            + [pltpu.SemaphoreType.DMA] * 6
        ),
        compiler_params=pltpu.CompilerParams(needs_layout_passes=False),
    )
    return run(x, s)


def kernel(x, score):
    return _sparsify(x, score)


# speedup vs baseline: 4.3909x; 1.1873x over previous
import jax
import jax.numpy as jnp
from jax import lax
from jax.experimental import pallas as pl
from jax.experimental.pallas import tpu as pltpu
from jax.experimental.pallas import tpu_sc as plsc

N_ROWS = 8192
N_COLS = 4096
BLK = 8
KEEP = 4

NC = 2
NS = 16
NW = NC * NS
PER_W = N_ROWS * N_COLS // NW
CHUNK = 16384
CW = CHUNK // 8
N_CHUNKS = PER_W // CHUNK
PANEL = 128
N_PANELS = CHUNK // PANEL


def _sc_body(x2_hbm, s2_hbm, o2_hbm,
             sb0, sb1, sb2, ob0, ob1, ob2,
             sx0, sx1, sx2, ss0, ss1, ss2, so0, so1, so2):
    x_hbm = x2_hbm.reshape(N_ROWS // 8, 8, N_COLS)
    s_hbm = s2_hbm.reshape(N_ROWS // 8, 8, N_COLS)
    o_hbm = o2_hbm.reshape(N_ROWS // 8, 8, N_COLS)
    wid = lax.axis_index("s") * NC + lax.axis_index("c")
    base_rc = wid * (N_CHUNKS // 2)

    lane8 = lax.iota(jnp.int32, 16) * 8

    sbufs, obufs = (sb0, sb1, sb2), (ob0, ob1, ob2)
    sxs, sss, sos = (sx0, sx1, sx2), (ss0, ss1, ss2), (so0, so1, so2)

    def in_copies(c, b):
        rc = base_rc + (c >> 1)
        ch = (c & 1) * CW
        return (
            pltpu.make_async_copy(
                x_hbm.at[rc, :, pl.ds(ch, CW)], obufs[b], sxs[b]),
            pltpu.make_async_copy(
                s_hbm.at[rc, :, pl.ds(ch, CW)], sbufs[b], sss[b]),
        )

    def out_copy(c, b):
        rc = base_rc + (c >> 1)
        ch = (c & 1) * CW
        return pltpu.make_async_copy(
            obufs[b], o_hbm.at[rc, :, pl.ds(ch, CW)], sos[b])

    def do_chunk(c, b):
        sb, ob = sbufs[b], obufs[b]

        cx, cs = in_copies(c, b)
        cx.wait()
        cs.wait()

        idx = [lane8 + k for k in range(BLK)]

        @plsc.parallel_loop(0, N_PANELS, unroll=2)
        def do_panel(p):
            r = p >> 4
            pbase = (p & 15) * PANEL
            sp = sb.at[r, pl.ds(pbase, PANEL)]
            op = ob.at[r, pl.ds(pbase, PANEL)]
            s = [plsc.load_gather(sp, [idx[k]]) for k in range(BLK)]
            terms = []
            for i in range(BLK):
                for j in range(i + 1, BLK):
                    kc = (1 << (4 * i)) - (1 << (4 * j))
                    terms.append(
                        jnp.where(s[j] < s[i], jnp.int32(kc), jnp.int32(0))
                    )
            while len(terms) > 1:
                terms = [
                    terms[t] + terms[t + 1] if t + 1 < len(terms) else terms[t]
                    for t in range(0, len(terms), 2)
                ]
            init = sum(k << (4 * k) for k in range(BLK))
            packed = jnp.full((16,), init, jnp.int32) + terms[0]
            zeros = jnp.zeros((16,), jnp.float32)
            for k in range(BLK):
                drop = (packed & (KEEP << (4 * k))) == 0
                plsc.store_scatter(op, [idx[k]], zeros, mask=drop)

        out_copy(c, b).start()

        b2 = (b + 2) % 3

        @pl.when(c + 2 < N_CHUNKS)
        def _():
            @pl.when(c >= 1)
            def _():
                out_copy(c - 1, b2).wait()
            nx, ns = in_copies(c + 2, b2)
            nx.start()
            ns.start()

    nx0, ns0 = in_copies(0, 0)
    nx0.start()
    ns0.start()
    nx1, ns1 = in_copies(1, 1)
    nx1.start()
    ns1.start()

    def tri_body(i, _):
        do_chunk(3 * i, 0)
        do_chunk(3 * i + 1, 1)
        do_chunk(3 * i + 2, 2)
        return ()

    lax.fori_loop(0, N_CHUNKS // 3, tri_body, ())
    for c in range((N_CHUNKS // 3) * 3, N_CHUNKS):
        do_chunk(c, c % 3)
    out_copy(N_CHUNKS - 3, (N_CHUNKS - 3) % 3).wait()
    out_copy(N_CHUNKS - 2, (N_CHUNKS - 2) % 3).wait()
    out_copy(N_CHUNKS - 1, (N_CHUNKS - 1) % 3).wait()


@jax.jit
def _sparsify(x, s):
    mesh = plsc.VectorSubcoreMesh(core_axis_name="c", subcore_axis_name="s")
    run = pl.kernel(
        _sc_body,
        mesh=mesh,
        out_type=jax.ShapeDtypeStruct((N_ROWS, N_COLS), jnp.float32),
        scratch_types=(
            [pltpu.VMEM((8, CW), jnp.float32)] * 6
            + [pltpu.SemaphoreType.DMA] * 9
        ),
        compiler_params=pltpu.CompilerParams(needs_layout_passes=False),
    )
    return run(x, s)


def kernel(x, score):
    return _sparsify(x, score)
